# baseline (speedup 1.0000x reference)
import functools

import jax
import jax.numpy as jnp
from jax import lax
from jax.experimental import pallas as pl
from jax.experimental.pallas import tpu as pltpu
from jax.experimental.pallas import tpu_sc as plsc

EMBED_DIM = 128
LANES = 16
CHUNK = 128
NBUF = 2


@functools.lru_cache(maxsize=None)
def _build_sc_kernel(batch: int, num_classes: int, dim: int):
    info = plsc.get_sparse_core_info()
    nc, ns = info.num_cores, info.num_subcores
    nw = nc * ns
    per_w = batch // nw
    if per_w >= 2 * CHUNK and per_w % CHUNK == 0:
        sizes = [CHUNK // 4, CHUNK // 4, CHUNK // 2] + [CHUNK] * (per_w // CHUNK - 1)
    else:
        sizes = [CHUNK] * (per_w // CHUNK)
    offs = [sum(sizes[:i]) for i in range(len(sizes))]
    nchunk = len(sizes)
    assert per_w * nw == batch and sum(sizes) == per_w and dim == EMBED_DIM

    mesh = plsc.VectorSubcoreMesh(core_axis_name="c", subcore_axis_name="s")

    @functools.partial(
        pl.kernel,
        mesh=mesh,
        compiler_params=pltpu.CompilerParams(needs_layout_passes=False),
        out_type=jax.ShapeDtypeStruct((batch,), jnp.float32),
        scratch_types=(
            [pltpu.VMEM((per_w,), jnp.int32)] * 2
            + [pltpu.VMEM((CHUNK, EMBED_DIM), jnp.float32)] * (2 * NBUF)
            + [pltpu.VMEM((per_w,), jnp.float32)]
            + [pltpu.SemaphoreType.DMA] * NBUF
        ),
    )
    def sc_kernel(s_hbm, d_hbm, w_hbm, out_hbm,
                  sid_v, did_v, *rest):
        sbufs = rest[0:NBUF]
        dbufs = rest[NBUF:2 * NBUF]
        out_v = rest[2 * NBUF]
        sems = rest[2 * NBUF + 1:]
        wid = lax.axis_index("s") * nc + lax.axis_index("c")
        base = wid * per_w

        iota = lax.iota(jnp.int32, LANES)
        last_lane = iota == (LANES - 1)
        zeros_i = jnp.zeros((LANES,), jnp.int32)
        inflight = [None] * nchunk

        def start_gather(c):
            b = c % NBUF
            n, o = sizes[c], offs[c]
            h1 = pltpu.async_copy(w_hbm.at[sid_v.at[pl.ds(o, n)]],
                                  sbufs[b].at[pl.ds(0, n)], sems[b])
            h2 = pltpu.async_copy(w_hbm.at[did_v.at[pl.ds(o, n)]],
                                  dbufs[b].at[pl.ds(0, n)], sems[b])
            inflight[c] = (h1, h2)

        def compute(c):
            b = c % NBUF
            sb, db = sbufs[b], dbufs[b]

            @plsc.parallel_loop(0, sizes[c], 1, unroll=1)
            def _(p):
                prods = [sb[p, pl.ds(k * LANES, LANES)] * db[p, pl.ds(k * LANES, LANES)]
                         for k in range(EMBED_DIM // LANES)]
                while len(prods) > 1:
                    prods = [a + b2 for a, b2 in zip(prods[::2], prods[1::2])]
                csum = plsc.cumsum(prods[0])
                res = 1.0 / (1.0 + jnp.exp(csum))
                plsc.store_scatter(out_v, [zeros_i + (offs[c] + p)], res,
                                   mask=last_lane)

        hs = pltpu.async_copy(s_hbm.at[pl.ds(base, per_w)], sid_v, sems[0])
        hd = pltpu.async_copy(d_hbm.at[pl.ds(base, per_w)], did_v, sems[1])
        hs.wait()
        hd.wait()

        depth = NBUF - 1
        for c in range(min(depth, nchunk)):
            start_gather(c)
        for c in range(nchunk):
            if c + depth < nchunk:
                start_gather(c + depth)
            with jax.named_scope("dma_wait"):
                for h in inflight[c]:
                    h.wait()
            with jax.named_scope("compute"):
                compute(c)
        pltpu.sync_copy(out_v, out_hbm.at[pl.ds(base, per_w)])

    return sc_kernel


def kernel(x, W):
    s = x[:, 0]
    d = x[:, 2]
    sck = _build_sc_kernel(x.shape[0], W.shape[0], W.shape[1])
    return sck(s, d, W)


# speedup vs baseline: 1.0207x; 1.0207x over previous
import functools

import jax
import jax.numpy as jnp
from jax import lax
from jax.experimental import pallas as pl
from jax.experimental.pallas import tpu as pltpu
from jax.experimental.pallas import tpu_sc as plsc

EMBED_DIM = 128
LANES = 16
CHUNK = 64
NBUF = 4


@functools.lru_cache(maxsize=None)
def _build_sc_kernel(batch: int, num_classes: int, dim: int):
    info = plsc.get_sparse_core_info()
    nc, ns = info.num_cores, info.num_subcores
    nw = nc * ns
    per_w = batch // nw
    sizes = [CHUNK] * (per_w // CHUNK)
    offs = [sum(sizes[:i]) for i in range(len(sizes))]
    nchunk = len(sizes)
    assert per_w * nw == batch and sum(sizes) == per_w and dim == EMBED_DIM

    mesh = plsc.VectorSubcoreMesh(core_axis_name="c", subcore_axis_name="s")

    @functools.partial(
        pl.kernel,
        mesh=mesh,
        compiler_params=pltpu.CompilerParams(needs_layout_passes=False),
        out_type=jax.ShapeDtypeStruct((batch,), jnp.float32),
        scratch_types=(
            [pltpu.VMEM((per_w,), jnp.int32)] * 2
            + [pltpu.VMEM((CHUNK, EMBED_DIM), jnp.float32)] * (2 * NBUF)
            + [pltpu.VMEM((per_w,), jnp.float32)]
            + [pltpu.SemaphoreType.DMA] * NBUF
        ),
    )
    def sc_kernel(s_hbm, d_hbm, w_hbm, out_hbm,
                  sid_v, did_v, *rest):
        sbufs = rest[0:NBUF]
        dbufs = rest[NBUF:2 * NBUF]
        out_v = rest[2 * NBUF]
        sems = rest[2 * NBUF + 1:]
        wid = lax.axis_index("s") * nc + lax.axis_index("c")
        base = wid * per_w

        iota = lax.iota(jnp.int32, LANES)
        last_lane = iota == (LANES - 1)
        zeros_i = jnp.zeros((LANES,), jnp.int32)
        inflight = [None] * nchunk

        def start_gather(c):
            b = c % NBUF
            n, o = sizes[c], offs[c]
            h1 = pltpu.async_copy(w_hbm.at[sid_v.at[pl.ds(o, n)]],
                                  sbufs[b].at[pl.ds(0, n)], sems[b])
            h2 = pltpu.async_copy(w_hbm.at[did_v.at[pl.ds(o, n)]],
                                  dbufs[b].at[pl.ds(0, n)], sems[b])
            inflight[c] = (h1, h2)

        def compute(c):
            b = c % NBUF
            sb, db = sbufs[b], dbufs[b]

            @plsc.parallel_loop(0, sizes[c], 1, unroll=1)
            def _(p):
                prods = [sb[p, pl.ds(k * LANES, LANES)] * db[p, pl.ds(k * LANES, LANES)]
                         for k in range(EMBED_DIM // LANES)]
                while len(prods) > 1:
                    prods = [a + b2 for a, b2 in zip(prods[::2], prods[1::2])]
                csum = plsc.cumsum(prods[0])
                res = 1.0 / (1.0 + jnp.exp(csum))
                plsc.store_scatter(out_v, [zeros_i + (offs[c] + p)], res,
                                   mask=last_lane)

        hs = pltpu.async_copy(s_hbm.at[pl.ds(base, per_w)], sid_v, sems[0])
        hd = pltpu.async_copy(d_hbm.at[pl.ds(base, per_w)], did_v, sems[1])
        hs.wait()
        hd.wait()

        depth = NBUF - 1
        for c in range(min(depth, nchunk)):
            start_gather(c)
        for c in range(nchunk):
            if c + depth < nchunk:
                start_gather(c + depth)
            with jax.named_scope("dma_wait"):
                for h in inflight[c]:
                    h.wait()
            with jax.named_scope("compute"):
                compute(c)
        pltpu.sync_copy(out_v, out_hbm.at[pl.ds(base, per_w)])

    return sc_kernel


def kernel(x, W):
    s = x[:, 0]
    d = x[:, 2]
    sck = _build_sc_kernel(x.shape[0], W.shape[0], W.shape[1])
    return sck(s, d, W)
